# baseline (speedup 1.0000x reference)
import functools

import jax
import jax.numpy as jnp
from jax import lax
from jax.experimental import pallas as pl
from jax.experimental.pallas import tpu as pltpu
from jax.experimental.pallas import tpu_sc as plsc

_N = 10000
_D = 128
_E = 320000

_NC = 2
_NS = 16
_DH = _D // _NC
_DW = 16
_CHUNK = 80
_DI = 4
_DG = 2
_RPT = _N // _NS
_EPT = _E // _NS
_NCH = _EPT // _CHUNK
_LOOP = (_NCH // _DI) * _DI
_BR = 1000
_BRP = 1000


def _tc_pre(x_ref, wcat_ref, g_ref, b_ref, bias_ref, z_ref, res_ref):
    xr = x_ref[...]
    mu = jnp.mean(xr, axis=1, keepdims=True)
    d = xr - mu
    var = jnp.mean(d * d, axis=1, keepdims=True)
    xn = d * lax.rsqrt(var + 1e-5) * g_ref[...] + b_ref[...]
    zz = lax.dot_general(xn, wcat_ref[...], (((1,), (0,)), ((), ())),
                         preferred_element_type=jnp.float32)
    res_ref[...] = zz[:, _D:] + xr + bias_ref[...]
    z_ref[0] = zz[:, :_DH]
    z_ref[1] = zz[:, _DH:_D]


def _tc_post(acc_ref, deg_ref, res_ref, out_ref):
    agg = jnp.concatenate([acc_ref[0], acc_ref[1]], axis=1)
    deg = deg_ref[0, :, 0:1]
    mean = agg / jnp.maximum(deg, 1.0)
    out_ref[...] = jnp.maximum(mean + res_ref[...], 0.0)


def _sc_body(z2_hbm, src_hbm, dst_hbm, zero_hbm, zerod_hbm, ones_hbm,
             out_hbm, outd_hbm,
             isrc_v, idst_v, rows_v, ones_v, z_sh, acc_sh, deg_sh,
             sem_is, sem_id, sem_g):
    c = lax.axis_index("c")
    s = lax.axis_index("s")
    base = s * _EPT
    pltpu.sync_copy(z2_hbm.at[c, pl.ds(s * _RPT, _RPT)],
                    z_sh.at[pl.ds(s * _RPT, _RPT)])
    pltpu.sync_copy(zero_hbm, acc_sh.at[pl.ds(s * _RPT, _RPT)])
    pltpu.sync_copy(zerod_hbm, deg_sh.at[pl.ds(s * _RPT, _RPT)])
    pltpu.sync_copy(ones_hbm, ones_v)
    plsc.subcore_barrier()

    def fetch_idx(j, u):
        pltpu.async_copy(src_hbm.at[pl.ds(base + j * _CHUNK, _CHUNK)],
                         isrc_v.at[u], sem_is.at[u])
        pltpu.async_copy(dst_hbm.at[pl.ds(base + j * _CHUNK, _CHUNK)],
                         idst_v.at[u], sem_id.at[u])

    def issue_gather(j, u):
        pltpu.make_async_copy(src_hbm.at[pl.ds(base + j * _CHUNK, _CHUNK)],
                              isrc_v.at[u % _DI], sem_is.at[u % _DI]).wait()
        pltpu.async_copy(z_sh.at[isrc_v.at[u % _DI]],
                         rows_v.at[u % _DG], sem_g.at[u % _DG])

    def consume(j, u):
        pltpu.make_async_copy(z_sh.at[isrc_v.at[u % _DI]],
                              rows_v.at[u % _DG], sem_g.at[u % _DG]).wait()
        pltpu.make_async_copy(dst_hbm.at[pl.ds(base + j * _CHUNK, _CHUNK)],
                              idst_v.at[u % _DI], sem_id.at[u % _DI]).wait()
        pltpu.sync_copy(rows_v.at[u % _DG],
                        acc_sh.at[idst_v.at[u % _DI]], add=True)
        pltpu.sync_copy(ones_v, deg_sh.at[idst_v.at[u % _DI]], add=True)

    for k in range(_DI):
        fetch_idx(k, k)
    for k in range(_DG):
        issue_gather(k, k)

    def body(t, carry):
        for u in range(_DI):
            j = t * _DI + u
            consume(j, u)

            @pl.when(j + _DI < _NCH)
            def _():
                fetch_idx(j + _DI, u)

            @pl.when(j + _DG < _NCH)
            def _():
                issue_gather(j + _DG, (u + _DG) % _DI)
        return carry

    lax.fori_loop(0, _LOOP // _DI, body, 0)
    for j in range(_LOOP, _NCH):
        consume(j, j % _DI)

    plsc.subcore_barrier()
    pltpu.sync_copy(acc_sh.at[pl.ds(s * _RPT, _RPT)],
                    out_hbm.at[c, pl.ds(s * _RPT, _RPT)])
    pltpu.sync_copy(deg_sh.at[pl.ds(s * _RPT, _RPT)],
                    outd_hbm.at[c, pl.ds(s * _RPT, _RPT)])


@functools.cache
def _sc_scatter():
    return pl.kernel(
        _sc_body,
        out_type=(
            jax.ShapeDtypeStruct((_NC, _N, _DH), jnp.float32),
            jax.ShapeDtypeStruct((_NC, _N, _DW), jnp.float32),
        ),
        mesh=plsc.VectorSubcoreMesh(core_axis_name="c", subcore_axis_name="s",
                                    num_cores=_NC, num_subcores=_NS),
        scratch_types=[
            pltpu.VMEM((_DI, _CHUNK), jnp.int32),
            pltpu.VMEM((_DI, _CHUNK), jnp.int32),
            pltpu.VMEM((_DG, _CHUNK, _DH), jnp.float32),
            pltpu.VMEM((_CHUNK, _DW), jnp.float32),
            pltpu.VMEM_SHARED((_N, _DH), jnp.float32),
            pltpu.VMEM_SHARED((_N, _DH), jnp.float32),
            pltpu.VMEM_SHARED((_N, _DW), jnp.float32),
            pltpu.SemaphoreType.DMA((_DI,)),
            pltpu.SemaphoreType.DMA((_DI,)),
            pltpu.SemaphoreType.DMA((_DG,)),
        ],
        compiler_params=pltpu.CompilerParams(use_tc_tiling_on_sc=False),
    )


def kernel(x, edge_index, edge_attr, h, batch, W_l, b_l, W_r, b_r, gamma, beta):
    wcat = jnp.concatenate([W_l.T, W_r.T], axis=1)
    bias = (b_l + b_r).reshape(1, _D)
    g = gamma.reshape(1, _D)
    b = beta.reshape(1, _D)

    z2, res = pl.pallas_call(
        _tc_pre,
        grid=(_N // _BR,),
        in_specs=[
            pl.BlockSpec((_BR, _D), lambda i: (i, 0)),
            pl.BlockSpec((_D, 2 * _D), lambda i: (0, 0)),
            pl.BlockSpec((1, _D), lambda i: (0, 0)),
            pl.BlockSpec((1, _D), lambda i: (0, 0)),
            pl.BlockSpec((1, _D), lambda i: (0, 0)),
        ],
        out_specs=[
            pl.BlockSpec((_NC, _BR, _DH), lambda i: (0, i, 0)),
            pl.BlockSpec((_BR, _D), lambda i: (i, 0)),
        ],
        out_shape=[
            jax.ShapeDtypeStruct((_NC, _N, _DH), jnp.float32),
            jax.ShapeDtypeStruct((_N, _D), jnp.float32),
        ],
    )(x, wcat, g, b, bias)

    zero = jnp.zeros((_RPT, _DH), jnp.float32)
    zerod = jnp.zeros((_RPT, _DW), jnp.float32)
    ones = jnp.zeros((_CHUNK, _DW), jnp.float32).at[:, 0].set(1.0)
    acc, dega = _sc_scatter()(z2, edge_index[0], edge_index[1],
                              zero, zerod, ones)

    out = pl.pallas_call(
        _tc_post,
        grid=(_N // _BRP,),
        in_specs=[
            pl.BlockSpec((_NC, _BRP, _DH), lambda i: (0, i, 0)),
            pl.BlockSpec((_NC, _BRP, _DW), lambda i: (0, i, 0)),
            pl.BlockSpec((_BRP, _D), lambda i: (i, 0)),
        ],
        out_specs=pl.BlockSpec((_BRP, _D), lambda i: (i, 0)),
        out_shape=jax.ShapeDtypeStruct((_N, _D), jnp.float32),
    )(acc, dega, res)

    return (out, h)


# speedup vs baseline: 1.2828x; 1.2828x over previous
import functools

import jax
import jax.numpy as jnp
from jax import lax
from jax.experimental import pallas as pl
from jax.experimental.pallas import tpu as pltpu
from jax.experimental.pallas import tpu_sc as plsc

_N = 10000
_D = 128
_E = 320000

_NC = 2
_NS = 16
_NW = _NC * _NS
_DW = 16
_CHUNK = 64
_NBUF = 2
_RPT = _N // _NS
_EPW = _E // _NW
_CPW = _EPW // _CHUNK
_TAIL = _EPW - _CPW * _CHUNK
_BR = 2000
_BRP = 2000


def _tc_pre(x_ref, wcat_ref, g_ref, b_ref, bias_ref, z_ref, res_ref):
    xr = x_ref[...]
    mu = jnp.mean(xr, axis=1, keepdims=True)
    d = xr - mu
    var = jnp.mean(d * d, axis=1, keepdims=True)
    xn = d * lax.rsqrt(var + 1e-5) * g_ref[...] + b_ref[...]
    zz = lax.dot_general(xn, wcat_ref[...], (((1,), (0,)), ((), ())),
                         preferred_element_type=jnp.float32)
    res_ref[...] = zz[:, _D:] + xr + bias_ref[...]
    z_ref[...] = zz[:, :_D]


def _tc_post(acc_ref, deg_ref, res_ref, out_ref):
    agg = acc_ref[0] + acc_ref[1]
    deg = deg_ref[0, :, 0:1] + deg_ref[1, :, 0:1]
    mean = agg / jnp.maximum(deg, 1.0)
    out_ref[...] = jnp.maximum(mean + res_ref[...], 0.0)


def _sc_body(z_hbm, src_hbm, dst_hbm, zero_hbm, zerod_hbm, ones_hbm,
             out_hbm, outd_hbm,
             src_v, dst_v, rows_v, ones_v, acc_sh, deg_sh, sems):
    c = lax.axis_index("c")
    s = lax.axis_index("s")
    wid = c * _NS + s
    pltpu.sync_copy(zero_hbm, acc_sh.at[pl.ds(s * _RPT, _RPT)])
    pltpu.sync_copy(zerod_hbm, deg_sh.at[pl.ds(s * _RPT, _RPT)])
    pltpu.sync_copy(src_hbm.at[pl.ds(wid * _EPW, _EPW)], src_v)
    pltpu.sync_copy(dst_hbm.at[pl.ds(wid * _EPW, _EPW)], dst_v)
    pltpu.sync_copy(ones_hbm, ones_v)
    plsc.subcore_barrier()

    for b in range(_NBUF):
        pltpu.async_copy(
            z_hbm.at[src_v.at[pl.ds(b * _CHUNK, _CHUNK)]],
            rows_v.at[b], sems.at[b])

    def body(t, carry):
        for b in range(_NBUF):
            j = t * _NBUF + b
            pltpu.make_async_copy(
                z_hbm.at[src_v.at[pl.ds(j * _CHUNK, _CHUNK)]],
                rows_v.at[b], sems.at[b]).wait()
            pltpu.sync_copy(rows_v.at[b],
                            acc_sh.at[dst_v.at[pl.ds(j * _CHUNK, _CHUNK)]],
                            add=True)
            pltpu.sync_copy(ones_v,
                            deg_sh.at[dst_v.at[pl.ds(j * _CHUNK, _CHUNK)]],
                            add=True)

            @pl.when(j + _NBUF < _CPW)
            def _():
                pltpu.async_copy(
                    z_hbm.at[src_v.at[pl.ds((j + _NBUF) * _CHUNK, _CHUNK)]],
                    rows_v.at[b], sems.at[b])
        return carry

    lax.fori_loop(0, _CPW // _NBUF, body, 0)
    pltpu.sync_copy(
        z_hbm.at[src_v.at[pl.ds(_CPW * _CHUNK, _TAIL)]],
        rows_v.at[0, pl.ds(0, _TAIL)])
    pltpu.sync_copy(rows_v.at[0, pl.ds(0, _TAIL)],
                    acc_sh.at[dst_v.at[pl.ds(_CPW * _CHUNK, _TAIL)]],
                    add=True)
    pltpu.sync_copy(ones_v.at[pl.ds(0, _TAIL)],
                    deg_sh.at[dst_v.at[pl.ds(_CPW * _CHUNK, _TAIL)]],
                    add=True)
    plsc.subcore_barrier()
    pltpu.sync_copy(acc_sh.at[pl.ds(s * _RPT, _RPT)],
                    out_hbm.at[c, pl.ds(s * _RPT, _RPT)])
    pltpu.sync_copy(deg_sh.at[pl.ds(s * _RPT, _RPT)],
                    outd_hbm.at[c, pl.ds(s * _RPT, _RPT)])


@functools.cache
def _sc_scatter():
    return pl.kernel(
        _sc_body,
        out_type=(
            jax.ShapeDtypeStruct((_NC, _N, _D), jnp.float32),
            jax.ShapeDtypeStruct((_NC, _N, _DW), jnp.float32),
        ),
        mesh=plsc.VectorSubcoreMesh(core_axis_name="c", subcore_axis_name="s",
                                    num_cores=_NC, num_subcores=_NS),
        scratch_types=[
            pltpu.VMEM((_EPW,), jnp.int32),
            pltpu.VMEM((_EPW,), jnp.int32),
            pltpu.VMEM((_NBUF, _CHUNK, _D), jnp.float32),
            pltpu.VMEM((_CHUNK, _DW), jnp.float32),
            pltpu.VMEM_SHARED((_N, _D), jnp.float32),
            pltpu.VMEM_SHARED((_N, _DW), jnp.float32),
            pltpu.SemaphoreType.DMA((_NBUF,)),
        ],
        compiler_params=pltpu.CompilerParams(use_tc_tiling_on_sc=False),
    )


def kernel(x, edge_index, edge_attr, h, batch, W_l, b_l, W_r, b_r, gamma, beta):
    wcat = jnp.concatenate([W_l.T, W_r.T], axis=1)
    bias = (b_l + b_r).reshape(1, _D)
    g = gamma.reshape(1, _D)
    b = beta.reshape(1, _D)

    z, res = pl.pallas_call(
        _tc_pre,
        grid=(_N // _BR,),
        in_specs=[
            pl.BlockSpec((_BR, _D), lambda i: (i, 0)),
            pl.BlockSpec((_D, 2 * _D), lambda i: (0, 0)),
            pl.BlockSpec((1, _D), lambda i: (0, 0)),
            pl.BlockSpec((1, _D), lambda i: (0, 0)),
            pl.BlockSpec((1, _D), lambda i: (0, 0)),
        ],
        out_specs=[
            pl.BlockSpec((_BR, _D), lambda i: (i, 0)),
            pl.BlockSpec((_BR, _D), lambda i: (i, 0)),
        ],
        out_shape=[
            jax.ShapeDtypeStruct((_N, _D), jnp.float32),
            jax.ShapeDtypeStruct((_N, _D), jnp.float32),
        ],
    )(x, wcat, g, b, bias)

    zero = jnp.zeros((_RPT, _D), jnp.float32)
    zerod = jnp.zeros((_RPT, _DW), jnp.float32)
    ones = jnp.zeros((_CHUNK, _DW), jnp.float32).at[:, 0].set(1.0)
    acc, dega = _sc_scatter()(z, edge_index[0], edge_index[1],
                              zero, zerod, ones)

    out = pl.pallas_call(
        _tc_post,
        grid=(_N // _BRP,),
        in_specs=[
            pl.BlockSpec((_NC, _BRP, _D), lambda i: (0, i, 0)),
            pl.BlockSpec((_NC, _BRP, _DW), lambda i: (0, i, 0)),
            pl.BlockSpec((_BRP, _D), lambda i: (i, 0)),
        ],
        out_specs=pl.BlockSpec((_BRP, _D), lambda i: (i, 0)),
        out_shape=jax.ShapeDtypeStruct((_N, _D), jnp.float32),
    )(acc, dega, res)

    return (out, h)
